# baseline (speedup 1.0000x reference)
import functools

import jax
import jax.numpy as jnp
from jax import lax
from jax.experimental import pallas as pl
from jax.experimental.pallas import tpu as pltpu
from jax.experimental.pallas import tpu_sc as plsc

NUM_EMB = 100
DIM = 50
PAD = 4
B0 = 16384
T = 200
NC, NS, L = 2, 16, 16
NW = NC * NS
TT = T // 8
BT = B0 // 128
NTILE = TT * BT
PER_W = NTILE // NW
NBUF = 2

_mesh = plsc.VectorSubcoreMesh(core_axis_name="c", subcore_axis_name="s")


@functools.partial(
    pl.kernel,
    mesh=_mesh,
    out_type=jax.ShapeDtypeStruct((DIM, T, B0), jnp.float32),
    scratch_types=[
        pltpu.VMEM((NUM_EMB * DIM,), jnp.float32),
        pltpu.VMEM((8, 128), jnp.int32),
        pltpu.VMEM((8, 128), jnp.int32),
        pltpu.VMEM((DIM, 8, 128), jnp.float32),
        pltpu.VMEM((DIM, 8, 128), jnp.float32),
        pltpu.SemaphoreType.DMA((NBUF,)),
        pltpu.SemaphoreType.DMA((NBUF,)),
        pltpu.SemaphoreType.DMA((NBUF,)),
    ],
    compiler_params=pltpu.CompilerParams(needs_layout_passes=False),
)
def _emb(idx_hbm, tab_hbm, out_hbm, tab_v, idx_v0, idx_v1, out_v0, out_v1,
         isem, osem, osem2):
    idx_bufs = (idx_v0, idx_v1)
    out_bufs = (out_v0, out_v1)
    wid = lax.axis_index("s") * NC + lax.axis_index("c")
    lane = lax.iota(jnp.int32, 16)

    pltpu.sync_copy(tab_hbm, tab_v)
    zero = jnp.zeros((16,), jnp.float32)
    for off in (0, 16, 32, 34):
        tab_v[pl.ds(PAD * DIM + off, 16)] = zero

    base_w = wid * PER_W

    def tile_coords(k):
        tau = base_w + k
        tt = tau // BT
        bb = tau % BT
        return tt * 8, bb * 128

    def idx_slice(k):
        t0, b0 = tile_coords(k)
        return idx_hbm.at[pl.ds(t0, 8), pl.ds(b0, 128)]

    H = DIM // 2

    def out_slice_lo(k):
        t0, b0 = tile_coords(k)
        return out_hbm.at[pl.ds(0, H), pl.ds(t0, 8), pl.ds(b0, 128)]

    def out_slice_hi(k):
        t0, b0 = tile_coords(k)
        return out_hbm.at[pl.ds(H, DIM - H), pl.ds(t0, 8), pl.ds(b0, 128)]

    for b in range(NBUF):
        pltpu.async_copy(idx_slice(b), idx_bufs[b], isem.at[b])

    def outer(cc, carry):
        for b in range(NBUF):
            k = cc * NBUF + b
            pltpu.make_async_copy(idx_slice(k), idx_bufs[b], isem.at[b]).wait()
            @pl.when(cc > 0)
            def _():
                pltpu.make_async_copy(
                    out_bufs[b].at[pl.ds(0, H)], out_slice_lo(k), osem.at[b]).wait()
                pltpu.make_async_copy(
                    out_bufs[b].at[pl.ds(H, DIM - H)], out_slice_hi(k), osem2.at[b]).wait()

            def vec_body(j, carry2):
                r = j // 8
                cb = (j % 8) * 16
                rv = jnp.zeros((16,), jnp.int32) + r
                cv = cb + lane
                idx16 = plsc.load_gather(idx_bufs[b], [rv, cv])
                g = idx16 * DIM
                K = 6
                pipe = []
                for d in range(DIM):
                    pipe.append(plsc.load_gather(tab_v, [g]))
                    g = g + 1
                    if len(pipe) > K:
                        ds = d - K
                        dv = jnp.full((16,), ds, jnp.int32)
                        plsc.store_scatter(out_bufs[b], [dv, rv, cv], pipe.pop(0))
                for ds in range(DIM - K, DIM):
                    dv = jnp.full((16,), ds, jnp.int32)
                    plsc.store_scatter(out_bufs[b], [dv, rv, cv], pipe.pop(0))
                return carry2

            lax.fori_loop(0, 64, vec_body, 0)
            pltpu.async_copy(out_bufs[b].at[pl.ds(0, H)], out_slice_lo(k), osem.at[b])
            pltpu.async_copy(out_bufs[b].at[pl.ds(H, DIM - H)], out_slice_hi(k), osem2.at[b])
            @pl.when(cc < PER_W // NBUF - 1)
            def _():
                pltpu.async_copy(idx_slice(k + NBUF), idx_bufs[b], isem.at[b])
        return carry

    lax.fori_loop(0, PER_W // NBUF, outer, 0)
    for b in range(NBUF):
        k = PER_W - NBUF + b
        pltpu.make_async_copy(
            out_bufs[b].at[pl.ds(0, H)], out_slice_lo(k), osem.at[b]).wait()
        pltpu.make_async_copy(
            out_bufs[b].at[pl.ds(H, DIM - H)], out_slice_hi(k), osem2.at[b]).wait()


def kernel(indices, table):
    idx_t = indices.astype(jnp.int32).T
    tab = table.reshape(-1)
    out = _emb(idx_t, tab)
    return out.transpose(2, 1, 0)


# speedup vs baseline: 1.0019x; 1.0019x over previous
import functools

import jax
import jax.numpy as jnp
from jax import lax
from jax.experimental import pallas as pl
from jax.experimental.pallas import tpu as pltpu
from jax.experimental.pallas import tpu_sc as plsc

NUM_EMB = 100
DIM = 50
PAD = 4
B0 = 16384
T = 200
NC, NS, L = 2, 16, 16
NW = NC * NS
TT = T // 8
BT = B0 // 128
NTILE = TT * BT
PER_W = NTILE // NW
NBUF = 2

_mesh = plsc.VectorSubcoreMesh(core_axis_name="c", subcore_axis_name="s")


@functools.partial(
    pl.kernel,
    mesh=_mesh,
    out_type=jax.ShapeDtypeStruct((DIM, T, B0), jnp.float32),
    scratch_types=[
        pltpu.VMEM((NUM_EMB * DIM,), jnp.float32),
        pltpu.VMEM((8, 128), jnp.int32),
        pltpu.VMEM((8, 128), jnp.int32),
        pltpu.VMEM((DIM, 8, 128), jnp.float32),
        pltpu.VMEM((DIM, 8, 128), jnp.float32),
        pltpu.SemaphoreType.DMA((NBUF,)),
        pltpu.SemaphoreType.DMA((NBUF,)),
    ],
    compiler_params=pltpu.CompilerParams(needs_layout_passes=False),
)
def _emb(idx_hbm, tab_hbm, out_hbm, tab_v, idx_v0, idx_v1, out_v0, out_v1,
         isem, osem):
    idx_bufs = (idx_v0, idx_v1)
    out_bufs = (out_v0, out_v1)
    wid = lax.axis_index("s") * NC + lax.axis_index("c")
    lane = lax.iota(jnp.int32, 16)

    pltpu.sync_copy(tab_hbm, tab_v)
    zero = jnp.zeros((16,), jnp.float32)
    for off in (0, 16, 32, 34):
        tab_v[pl.ds(PAD * DIM + off, 16)] = zero

    base_w = wid * PER_W

    def tile_coords(k):
        tau = base_w + k
        tt = tau // BT
        bb = tau % BT
        return tt * 8, bb * 128

    def idx_slice(k):
        t0, b0 = tile_coords(k)
        return idx_hbm.at[pl.ds(t0, 8), pl.ds(b0, 128)]

    def out_slice(k):
        t0, b0 = tile_coords(k)
        return out_hbm.at[pl.ds(0, DIM), pl.ds(t0, 8), pl.ds(b0, 128)]

    for b in range(NBUF):
        pltpu.async_copy(idx_slice(b), idx_bufs[b], isem.at[b])

    def outer(cc, carry):
        for b in range(NBUF):
            k = cc * NBUF + b
            pltpu.make_async_copy(idx_slice(k), idx_bufs[b], isem.at[b]).wait()
            @pl.when(cc > 0)
            def _():
                pltpu.make_async_copy(out_bufs[b], out_slice(k), osem.at[b]).wait()

            def vec_body(j, carry2):
                r = j // 8
                cb = (j % 8) * 16
                rv = jnp.zeros((16,), jnp.int32) + r
                cv = cb + lane
                idx16 = plsc.load_gather(idx_bufs[b], [rv, cv])
                g = idx16 * DIM
                K = 6
                pipe = []
                for d in range(DIM):
                    pipe.append(plsc.load_gather(tab_v, [g]))
                    g = g + 1
                    if len(pipe) > K:
                        ds = d - K
                        dv = jnp.full((16,), ds, jnp.int32)
                        plsc.store_scatter(out_bufs[b], [dv, rv, cv], pipe.pop(0))
                for ds in range(DIM - K, DIM):
                    dv = jnp.full((16,), ds, jnp.int32)
                    plsc.store_scatter(out_bufs[b], [dv, rv, cv], pipe.pop(0))
                return carry2

            lax.fori_loop(0, 64, vec_body, 0)
            pltpu.async_copy(out_bufs[b], out_slice(k), osem.at[b])
            @pl.when(cc < PER_W // NBUF - 1)
            def _():
                pltpu.async_copy(idx_slice(k + NBUF), idx_bufs[b], isem.at[b])
        return carry

    lax.fori_loop(0, PER_W // NBUF, outer, 0)
    for b in range(NBUF):
        k = PER_W - NBUF + b
        pltpu.make_async_copy(out_bufs[b], out_slice(k), osem.at[b]).wait()


def kernel(indices, table):
    idx_t = indices.astype(jnp.int32).T
    tab = table.reshape(-1)
    out = _emb(idx_t, tab)
    return out.transpose(2, 1, 0)
